# baseline (speedup 1.0000x reference)
import functools

import jax
import jax.numpy as jnp
from jax import lax
from jax.experimental import pallas as pl
from jax.experimental.pallas import tpu as pltpu
from jax.experimental.pallas import tpu_sc as plsc

T = 2048
D = 1024
E = 8
K = 2
TK = T * K
RB = 256
NB = TK // RB
NT = NB + E - 1
NC, NS = 2, 16
NW = NC * NS


def _gating_kernel(x_ref, gw_ref, gb_ref, idx_ref, wgt_ref):
    xc = x_ref[...]
    logits = jax.lax.dot_general(
        xc, gw_ref[...], (((1,), (1,)), ((), ())),
        preferred_element_type=jnp.float32,
        precision=jax.lax.Precision.DEFAULT,
    ) + gb_ref[...]
    m = jnp.max(logits, axis=1, keepdims=True)
    ex = jnp.exp(logits - m)
    w = ex / jnp.sum(ex, axis=1, keepdims=True)
    col = jax.lax.broadcasted_iota(jnp.int32, w.shape, 1)
    rank = jnp.zeros(w.shape, jnp.int32)
    for ep in range(E):
        wp = w[:, ep:ep + 1]
        rank = rank + (wp > w).astype(jnp.int32)
        rank = rank + ((wp == w) & (ep < col)).astype(jnp.int32)
    sel = rank < K
    e0 = jnp.min(jnp.where(sel, col, E), axis=1, keepdims=True)
    e1 = jnp.max(jnp.where(sel, col, -1), axis=1, keepdims=True)
    w0 = jnp.sum(jnp.where(col == e0, w, 0.0), axis=1, keepdims=True)
    w1 = jnp.sum(jnp.where(col == e1, w, 0.0), axis=1, keepdims=True)
    idx_ref[...] = jnp.concatenate([e0, e1], axis=1)
    wgt_ref[...] = jnp.concatenate([w0, w1], axis=1)


def _gating(x, gate_W, gate_b):
    return pl.pallas_call(
        _gating_kernel,
        out_shape=(jax.ShapeDtypeStruct((T, K), jnp.int32),
                   jax.ShapeDtypeStruct((T, K), jnp.float32)),
    )(x, gate_W, gate_b.reshape(1, E))


_CH = 16


def _make_sc_gather(n_rows):
    per_w = n_rows // NW
    mesh = plsc.VectorSubcoreMesh(core_axis_name="c", subcore_axis_name="s")

    @functools.partial(
        pl.kernel, mesh=mesh,
        out_type=jax.ShapeDtypeStruct((n_rows, D), jnp.float32),
        scratch_types=[
            pltpu.VMEM((_CH,), jnp.int32),
            pltpu.VMEM((_CH, D), jnp.float32),
            pltpu.SemaphoreType.DMA,
        ],
    )
    def gather_k(table_hbm, idx_hbm, out_hbm, idx_v, rows_v, sem):
        wid = lax.axis_index("s") * NC + lax.axis_index("c")
        base = wid * per_w

        def body(i, _):
            b0 = base + i * _CH
            pltpu.sync_copy(idx_hbm.at[pl.ds(b0, _CH)], idx_v)
            pltpu.async_copy(table_hbm.at[idx_v], rows_v, sem).wait()
            pltpu.sync_copy(rows_v, out_hbm.at[pl.ds(b0, _CH)])
            return 0

        lax.fori_loop(0, per_w // _CH, body, 0)

    return gather_k


def _make_sc_combine():
    per_w = T // NW
    mesh = plsc.VectorSubcoreMesh(core_axis_name="c", subcore_axis_name="s")

    @functools.partial(
        pl.kernel, mesh=mesh,
        out_type=jax.ShapeDtypeStruct((T, D), jnp.float32),
        scratch_types=[
            pltpu.VMEM((_CH,), jnp.int32),
            pltpu.VMEM((_CH,), jnp.int32),
            pltpu.VMEM((_CH, D), jnp.float32),
            pltpu.VMEM((_CH, D), jnp.float32),
            pltpu.SemaphoreType.DMA,
            pltpu.SemaphoreType.DMA,
        ],
    )
    def combine_k(y_hbm, pos0_hbm, pos1_hbm, out_hbm,
                  i0_v, i1_v, a_v, b_v, sem0, sem1):
        wid = lax.axis_index("s") * NC + lax.axis_index("c")
        base = wid * per_w

        def body(i, _):
            b0 = base + i * _CH
            pltpu.sync_copy(pos0_hbm.at[pl.ds(b0, _CH)], i0_v)
            pltpu.sync_copy(pos1_hbm.at[pl.ds(b0, _CH)], i1_v)
            pltpu.async_copy(y_hbm.at[i0_v], a_v, sem0).wait()
            pltpu.async_copy(y_hbm.at[i1_v], b_v, sem1).wait()
            for r in range(_CH):
                for cchunk in range(D // 16):
                    cs = pl.ds(cchunk * 16, 16)
                    a_v[r, cs] = a_v[r, cs] + b_v[r, cs]
            pltpu.sync_copy(a_v, out_hbm.at[pl.ds(b0, _CH)])
            return 0

        lax.fori_loop(0, per_w // _CH, body, 0)

    return combine_k


def _grouped_kernel(blk_s, exp_s, first_s, lo_s, hi_s,
                    xs_ref, ws_ref, ew_ref, eb_ref, y_ref):
    j = pl.program_id(0)
    xb = xs_ref[...].astype(jnp.bfloat16)
    wb = ew_ref[0].astype(jnp.bfloat16)
    y = jax.lax.dot_general(
        xb, wb, (((1,), (1,)), ((), ())),
        preferred_element_type=jnp.float32,
        precision=jax.lax.Precision.DEFAULT,
    ) + eb_ref[0]
    rid = jax.lax.broadcasted_iota(jnp.int32, (RB, 1), 0) + blk_s[j] * RB
    msk = ((rid >= lo_s[j]) & (rid < hi_s[j])).astype(jnp.float32)
    contrib = y * (ws_ref[...] * msk)

    @pl.when(first_s[j] == 1)
    def _init():
        y_ref[...] = contrib

    @pl.when(first_s[j] == 0)
    def _acc():
        y_ref[...] += contrib


def _grouped(x_sorted, w_sorted, expert_W, expert_b,
             blk, exp, first, lo, hi):
    grid_spec = pltpu.PrefetchScalarGridSpec(
        num_scalar_prefetch=5,
        grid=(NT,),
        in_specs=[
            pl.BlockSpec((RB, D),
                         lambda j, bs, es, fs, ls, hs: (bs[j], 0)),
            pl.BlockSpec((RB, 1),
                         lambda j, bs, es, fs, ls, hs: (bs[j], 0)),
            pl.BlockSpec((1, D, D),
                         lambda j, bs, es, fs, ls, hs: (es[j], 0, 0)),
            pl.BlockSpec((1, 1, D),
                         lambda j, bs, es, fs, ls, hs: (es[j], 0, 0)),
        ],
        out_specs=pl.BlockSpec((RB, D),
                               lambda j, bs, es, fs, ls, hs: (bs[j], 0)),
    )
    return pl.pallas_call(
        _grouped_kernel,
        grid_spec=grid_spec,
        out_shape=jax.ShapeDtypeStruct((TK, D), jnp.float32),
        compiler_params=pltpu.CompilerParams(
            dimension_semantics=("arbitrary",),
        ),
    )(blk, exp, first, lo, hi,
      x_sorted, w_sorted.reshape(TK, 1), expert_W,
      expert_b.reshape(E, 1, D))


def kernel(x, gate_W, gate_b, expert_W, expert_b):
    idx, wgt = _gating(x, gate_W, gate_b)

    e_flat = idx.reshape(-1)
    sort_idx = jnp.argsort(e_flat, stable=True)
    tok_sorted = (sort_idx // K).astype(jnp.int32)
    w_sorted = wgt.reshape(-1)[sort_idx]
    pos = jnp.zeros((TK,), jnp.int32).at[sort_idx].set(
        jnp.arange(TK, dtype=jnp.int32))
    pos0 = pos[0::2]
    pos1 = pos[1::2]
    counts = jnp.sum(
        (e_flat[None, :] == jnp.arange(E, dtype=e_flat.dtype)[:, None])
        .astype(jnp.int32), axis=1)
    off = jnp.concatenate(
        [jnp.zeros((1,), jnp.int32), jnp.cumsum(counts)]).astype(jnp.int32)
    b_ids = jnp.repeat(jnp.arange(NB, dtype=jnp.int32), E)
    e_ids = jnp.tile(jnp.arange(E, dtype=jnp.int32), NB)
    lo_all = jnp.maximum(off[e_ids], b_ids * RB)
    hi_all = jnp.minimum(off[e_ids + 1], (b_ids + 1) * RB)
    nonempty = hi_all > lo_all
    order = jnp.argsort(~nonempty, stable=True)[:NT]
    ne = nonempty[order]
    last_e = e_flat[sort_idx[TK - 1]].astype(jnp.int32)
    blk = jnp.where(ne, b_ids[order], NB - 1)
    exp = jnp.where(ne, e_ids[order], last_e)
    lo = jnp.where(ne, lo_all[order], 0)
    hi = jnp.where(ne, hi_all[order], 0)
    first = jnp.concatenate(
        [jnp.ones((1,), jnp.int32),
         (blk[1:] != blk[:-1]).astype(jnp.int32)])

    x_sorted = _make_sc_gather(TK)(x, tok_sorted)
    y_sorted = _grouped(x_sorted, w_sorted, expert_W, expert_b,
                        blk, exp, first, lo, hi)
    out = _make_sc_combine()(y_sorted, pos0, pos1)
    return out


# speedup vs baseline: 2.5621x; 2.5621x over previous
import jax
import jax.numpy as jnp
from jax.experimental import pallas as pl
from jax.experimental.pallas import tpu as pltpu

NUM_EXPERTS = 8
TOP_K = 2
CHUNK = 1024


def _moe_kernel(x_ref, gw_ref, gb_ref, ew_ref, eb_ref, out_ref,
                c_ref, xb_ref, wb_ref, acc_ref):
    e = pl.program_id(0)
    tb = pl.program_id(1)
    E = NUM_EXPERTS
    sl = pl.ds(tb * CHUNK, CHUNK)

    @pl.when(tb == 0)
    def _cast_w():
        wb_ref[...] = ew_ref[0].astype(jnp.bfloat16)

    @pl.when(e == 0)
    def _gating():
        xc = x_ref[...]
        logits = jax.lax.dot_general(
            xc, gw_ref[...], (((1,), (1,)), ((), ())),
            preferred_element_type=jnp.float32,
            precision=jax.lax.Precision.DEFAULT,
        ) + gb_ref[...]
        m = jnp.max(logits, axis=1, keepdims=True)
        ex = jnp.exp(logits - m)
        w = ex / jnp.sum(ex, axis=1, keepdims=True)
        col = jax.lax.broadcasted_iota(jnp.int32, w.shape, 1)
        rank = jnp.zeros(w.shape, jnp.int32)
        for ep in range(E):
            wp = w[:, ep:ep + 1]
            rank = rank + (wp > w).astype(jnp.int32)
            rank = rank + ((wp == w) & (ep < col)).astype(jnp.int32)
        c_ref[sl, :] = jnp.where(rank < TOP_K, w, 0.0)
        xb_ref[sl, :] = xc.astype(jnp.bfloat16)

    cc = c_ref[sl, :]
    ce = jnp.sum(
        jnp.where(
            jax.lax.broadcasted_iota(jnp.int32, cc.shape, 1) == e,
            cc, 0.0),
        axis=1, keepdims=True)
    y = jax.lax.dot_general(
        xb_ref[sl, :], wb_ref[...], (((1,), (1,)), ((), ())),
        preferred_element_type=jnp.float32,
        precision=jax.lax.Precision.DEFAULT,
    ) + eb_ref[0]
    contrib = ce * y

    @pl.when(e == 0)
    def _init():
        acc_ref[sl, :] = contrib

    @pl.when((e > 0) & (e < E - 1))
    def _acc():
        acc_ref[sl, :] += contrib

    @pl.when(e == E - 1)
    def _flush():
        out_ref[...] = acc_ref[sl, :] + contrib


def kernel(x, gate_W, gate_b, expert_W, expert_b):
    T, D = x.shape
    E = gate_W.shape[0]
    nb = T // CHUNK
    return pl.pallas_call(
        _moe_kernel,
        grid=(E, nb),
        in_specs=[
            pl.BlockSpec((CHUNK, D),
                         lambda e, tb: (jnp.where(e == 0, tb, 0), 0)),
            pl.BlockSpec((E, D), lambda e, tb: (0, 0)),
            pl.BlockSpec((1, E), lambda e, tb: (0, 0)),
            pl.BlockSpec((1, D, D), lambda e, tb: (e, 0, 0)),
            pl.BlockSpec((1, 1, D), lambda e, tb: (e, 0, 0)),
        ],
        out_specs=pl.BlockSpec(
            (CHUNK, D),
            lambda e, tb: (jnp.where(e == NUM_EXPERTS - 1, tb, 0), 0)),
        out_shape=jax.ShapeDtypeStruct((T, D), jnp.float32),
        scratch_shapes=[
            pltpu.VMEM((T, E), jnp.float32),
            pltpu.VMEM((T, D), jnp.bfloat16),
            pltpu.VMEM((D, D), jnp.bfloat16),
            pltpu.VMEM((T, D), jnp.float32),
        ],
        compiler_params=pltpu.CompilerParams(
            dimension_semantics=("arbitrary", "arbitrary"),
        ),
    )(x, gate_W, gate_b.reshape(1, E), expert_W, expert_b.reshape(E, 1, D))
